# baseline (speedup 1.0000x reference)
import functools

import jax
import jax.numpy as jnp
from jax import lax
from jax.experimental import pallas as pl
from jax.experimental.pallas import tpu as pltpu
from jax.experimental.pallas import tpu_sc as plsc

SEQ = 200
EMB = 64
PADDED = 128
BATCH = 1024
VOCAB = 100000
NC = 2
NS = 16
NW = NC * NS
LANES = 16
SCALE = 8.0

STR = SEQ // 8
MAXTR = 7
S_CHUNK = 2
ROWS = S_CHUNK * PADDED


def _sc_embed(idx4, tblv, pos_table):
    mesh = plsc.VectorSubcoreMesh(
        core_axis_name="c", subcore_axis_name="s", num_cores=NC, num_subcores=NS
    )

    @functools.partial(
        pl.kernel,
        mesh=mesh,
        compiler_params=pltpu.CompilerParams(
            use_tc_tiling_on_sc=False, needs_layout_passes=False
        ),
        out_type=jax.ShapeDtypeStruct((SEQ, 8, 8, 8, PADDED), jnp.float32),
        scratch_types=[
            pltpu.VMEM((MAXTR, 8, PADDED), jnp.int32),
            pltpu.VMEM((MAXTR * 8 * PADDED,), jnp.int32),
            pltpu.VMEM((SEQ, EMB), jnp.float32),
            pltpu.VMEM((ROWS, EMB), jnp.float32),
            pltpu.VMEM((ROWS, EMB), jnp.float32),
            pltpu.VMEM((S_CHUNK, 8, 8, PADDED), jnp.float32),
            pltpu.VMEM((S_CHUNK, 8, 8, PADDED), jnp.float32),
            pltpu.SemaphoreType.DMA,
            pltpu.SemaphoreType.DMA,
            pltpu.SemaphoreType.DMA,
            pltpu.SemaphoreType.DMA,
            pltpu.SemaphoreType.DMA,
        ],
    )
    def k(idx_hbm, tok_hbm, pos_hbm, out_hbm, idx_st, idx2_v, pos_v,
          gb0, gb1, ob0, ob1, ssem, gs0, gs1, ws0, ws1):
        gbufs = (gb0, gb1)
        obufs = (ob0, ob1)
        gsem = (gs0, gs1)
        wsem = (ws0, ws1)
        wid = lax.axis_index("s") * NC + lax.axis_index("c")
        tb = wid // 4
        q = wid % 4
        ntr = jnp.where(q == 0, 7, 6)
        tr0 = jnp.where(q == 0, 0, 7 + (q - 1) * 6)
        nch = ntr * 4

        pltpu.sync_copy(pos_hbm, pos_v)

        def idx_dma(t, carry):
            pltpu.async_copy(idx_hbm.at[tr0 + t, tb], idx_st.at[t], ssem)
            return carry

        lax.fori_loop(0, ntr, idx_dma, 0)

        def drain_idx(t, carry):
            pltpu.make_async_copy(idx_hbm.at[0, 0], idx_st.at[0], ssem).wait()
            return carry

        lax.fori_loop(0, ntr, drain_idx, 0)

        def idx_flat(t, carry):
            for sl in range(8):
                for h in range(PADDED // LANES):
                    off = t * 1024 + sl * PADDED + h * LANES
                    idx2_v[pl.ds(off, LANES)] = (
                        idx_st[t, sl, pl.ds(h * LANES, LANES)] * 2
                    )
            return carry

        lax.fori_loop(0, ntr, idx_flat, 0)

        def start_gather(kc, b):
            for h in range(2):
                pltpu.async_copy(
                    tok_hbm.at[idx2_v.at[pl.ds(kc * ROWS + h * 128, 128)]],
                    gbufs[b].at[pl.ds(h * 128, 128)],
                    gsem[b],
                )

        def wait_gather(b):
            pltpu.make_async_copy(tok_hbm.at[pl.ds(0, ROWS)], gbufs[b], gsem[b]).wait()

        def start_wb(kc, b):
            s0 = (tr0 + kc // 4) * 8 + (kc % 4) * S_CHUNK
            pltpu.async_copy(
                obufs[b],
                out_hbm.at[pl.ds(s0, S_CHUNK), :, tb, :, :],
                wsem[b],
            )

        def wait_wb(b):
            pltpu.make_async_copy(
                obufs[b],
                out_hbm.at[pl.ds(0, S_CHUNK), :, 0, :, :],
                wsem[b],
            ).wait()

        iota = lax.iota(jnp.int32, LANES)
        c_td = [(j * LANES + iota) >> 3 for j in range(EMB // LANES)]
        c_sd = [(j * LANES + iota) & 7 for j in range(EMB // LANES)]

        def compute(kc, b):
            gbuf = gbufs[b]
            obuf = obufs[b]
            s0 = (tr0 + kc // 4) * 8 + (kc % 4) * S_CHUNK
            for ss in range(S_CHUNK):
                prow = s0 + ss
                pos_j = [
                    pos_v[prow, pl.ds(j * LANES, LANES)]
                    for j in range(EMB // LANES)
                ]
                c_ss = jnp.full((LANES,), ss, jnp.int32)

                def b_body(bv, carry):
                    row = ss * PADDED + bv
                    i_b = jnp.full((LANES,), bv, jnp.int32)
                    for j in range(EMB // LANES):
                        v = gbuf[row, pl.ds(j * LANES, LANES)] * SCALE + pos_j[j]
                        plsc.store_scatter(obuf, [c_ss, c_td[j], c_sd[j], i_b], v)
                    return carry

                lax.fori_loop(0, PADDED, b_body, 0)

        start_gather(0, 0)

        def outer(o, carry):
            for phase in range(2):
                kc = 2 * o + phase
                b = phase

                @pl.when(kc + 1 < nch)
                def _():
                    start_gather(kc + 1, 1 - phase)

                wait_gather(b)

                @pl.when(kc >= 2)
                def _():
                    wait_wb(b)

                compute(kc, b)
                start_wb(kc, b)
            return carry

        lax.fori_loop(0, nch // 2, outer, 0)
        wait_wb(0)
        wait_wb(1)

    return k(idx4, tblv, pos_table)


def kernel(inputs, token_table, pos_table):
    idx4 = (
        inputs.astype(jnp.int32)
        .T.reshape(STR, 8, 8, PADDED)
        .transpose(0, 2, 1, 3)
    )
    tbl128 = jnp.pad(token_table.astype(jnp.float32), ((0, 0), (0, PADDED - EMB)))
    tblv = tbl128.reshape(2 * VOCAB, EMB)
    out5 = _sc_embed(idx4, tblv, pos_table.astype(jnp.float32))
    return jnp.transpose(out5, (2, 4, 0, 1, 3)).reshape(BATCH, SEQ, EMB)


# speedup vs baseline: 1.0043x; 1.0043x over previous
import functools

import jax
import jax.numpy as jnp
from jax import lax
from jax.experimental import pallas as pl
from jax.experimental.pallas import tpu as pltpu
from jax.experimental.pallas import tpu_sc as plsc

SEQ = 200
EMB = 64
PADDED = 128
BATCH = 1024
VOCAB = 100000
NC = 2
NS = 16
NW = NC * NS
LANES = 16
SCALE = 8.0

STR = SEQ // 8
MAXTR = 7
S_CHUNK = 2
ROWS = S_CHUNK * PADDED


def _sc_embed(idx4, tblv, pos_table):
    mesh = plsc.VectorSubcoreMesh(
        core_axis_name="c", subcore_axis_name="s", num_cores=NC, num_subcores=NS
    )

    @functools.partial(
        pl.kernel,
        mesh=mesh,
        compiler_params=pltpu.CompilerParams(
            use_tc_tiling_on_sc=False, needs_layout_passes=False
        ),
        out_type=jax.ShapeDtypeStruct((SEQ, 8, 8, 8, PADDED), jnp.float32),
        scratch_types=[
            pltpu.VMEM((MAXTR, 8, PADDED), jnp.int32),
            pltpu.VMEM((MAXTR * 8 * PADDED,), jnp.int32),
            pltpu.VMEM((SEQ, EMB), jnp.float32),
            pltpu.VMEM((ROWS, EMB), jnp.float32),
            pltpu.VMEM((ROWS, EMB), jnp.float32),
            pltpu.VMEM((S_CHUNK, 8, 8, PADDED), jnp.float32),
            pltpu.VMEM((S_CHUNK, 8, 8, PADDED), jnp.float32),
            pltpu.SemaphoreType.DMA,
            pltpu.SemaphoreType.DMA,
            pltpu.SemaphoreType.DMA,
            pltpu.SemaphoreType.DMA,
            pltpu.SemaphoreType.DMA,
        ],
    )
    def k(idx_hbm, tok_hbm, pos_hbm, out_hbm, idx_st, idx2_v, pos_v,
          gb0, gb1, ob0, ob1, ssem, gs0, gs1, ws0, ws1):
        gbufs = (gb0, gb1)
        obufs = (ob0, ob1)
        gsem = (gs0, gs1)
        wsem = (ws0, ws1)
        wid = lax.axis_index("s") * NC + lax.axis_index("c")
        tb = wid // 4
        q = wid % 4
        ntr = jnp.where(q == 0, 7, 6)
        tr0 = jnp.where(q == 0, 0, 7 + (q - 1) * 6)
        nch = ntr * 4

        pltpu.sync_copy(pos_hbm, pos_v)

        def idx_dma(t, carry):
            pltpu.async_copy(idx_hbm.at[tr0 + t, tb], idx_st.at[t], ssem)
            return carry

        lax.fori_loop(0, ntr, idx_dma, 0)

        def drain_idx(t, carry):
            pltpu.make_async_copy(idx_hbm.at[0, 0], idx_st.at[0], ssem).wait()
            return carry

        lax.fori_loop(0, ntr, drain_idx, 0)

        def idx_flat(t, carry):
            for sl in range(8):
                for h in range(PADDED // LANES):
                    off = t * 1024 + sl * PADDED + h * LANES
                    idx2_v[pl.ds(off, LANES)] = (
                        idx_st[t, sl, pl.ds(h * LANES, LANES)] * 2
                    )
            return carry

        lax.fori_loop(0, ntr, idx_flat, 0)

        def start_gather(kc, b):
            for h in range(2):
                pltpu.async_copy(
                    tok_hbm.at[idx2_v.at[pl.ds(kc * ROWS + h * 128, 128)]],
                    gbufs[b].at[pl.ds(h * 128, 128)],
                    gsem[b],
                )

        def wait_gather(b):
            pltpu.make_async_copy(tok_hbm.at[pl.ds(0, ROWS)], gbufs[b], gsem[b]).wait()

        def start_wb(kc, b):
            s0 = (tr0 + kc // 4) * 8 + (kc % 4) * S_CHUNK
            pltpu.async_copy(
                obufs[b],
                out_hbm.at[pl.ds(s0, S_CHUNK), :, tb, :, :],
                wsem[b],
            )

        def wait_wb(b):
            pltpu.make_async_copy(
                obufs[b],
                out_hbm.at[pl.ds(0, S_CHUNK), :, 0, :, :],
                wsem[b],
            ).wait()

        iota = lax.iota(jnp.int32, LANES)
        c_td = [(j * LANES + iota) >> 3 for j in range(EMB // LANES)]
        c_sd = [(j * LANES + iota) & 7 for j in range(EMB // LANES)]

        def compute(kc, b):
            gbuf = gbufs[b]
            obuf = obufs[b]
            s0 = (tr0 + kc // 4) * 8 + (kc % 4) * S_CHUNK
            for ss in range(S_CHUNK):
                prow = s0 + ss
                pos_j = [
                    pos_v[prow, pl.ds(j * LANES, LANES)]
                    for j in range(EMB // LANES)
                ]
                c_ss = jnp.full((LANES,), ss, jnp.int32)

                def b_body(bq, carry):
                    for u in range(4):
                        bv = bq * 4 + u
                        row = ss * PADDED + bv
                        i_b = jnp.full((LANES,), bv, jnp.int32)
                        for j in range(EMB // LANES):
                            v = gbuf[row, pl.ds(j * LANES, LANES)] * SCALE \
                                + pos_j[j]
                            plsc.store_scatter(
                                obuf, [c_ss, c_td[j], c_sd[j], i_b], v
                            )
                    return carry

                lax.fori_loop(0, PADDED // 4, b_body, 0)

        start_gather(0, 0)

        def outer(o, carry):
            for phase in range(2):
                kc = 2 * o + phase
                b = phase

                @pl.when(kc + 1 < nch)
                def _():
                    start_gather(kc + 1, 1 - phase)

                wait_gather(b)

                @pl.when(kc >= 2)
                def _():
                    wait_wb(b)

                compute(kc, b)
                start_wb(kc, b)
            return carry

        lax.fori_loop(0, nch // 2, outer, 0)
        wait_wb(0)
        wait_wb(1)

    return k(idx4, tblv, pos_table)


def kernel(inputs, token_table, pos_table):
    idx4 = (
        inputs.astype(jnp.int32)
        .T.reshape(STR, 8, 8, PADDED)
        .transpose(0, 2, 1, 3)
    )
    tbl128 = jnp.pad(token_table.astype(jnp.float32), ((0, 0), (0, PADDED - EMB)))
    tblv = tbl128.reshape(2 * VOCAB, EMB)
    out5 = _sc_embed(idx4, tblv, pos_table.astype(jnp.float32))
    return jnp.transpose(out5, (2, 4, 0, 1, 3)).reshape(BATCH, SEQ, EMB)


# speedup vs baseline: 1.6589x; 1.6519x over previous
import functools

import jax
import jax.numpy as jnp
from jax import lax
from jax.experimental import pallas as pl
from jax.experimental.pallas import tpu as pltpu
from jax.experimental.pallas import tpu_sc as plsc

SEQ = 200
EMB = 64
PADDED = 128
BATCH = 1024
VOCAB = 100000
NC = 2
NS = 16
NW = NC * NS
LANES = 16
SCALE = 8.0

STR = SEQ // 8
MAXTR = 7
S_CHUNK = 2
ROWS = S_CHUNK * PADDED


def _sc_embed(idx4, tblv, pos_table):
    mesh = plsc.VectorSubcoreMesh(
        core_axis_name="c", subcore_axis_name="s", num_cores=NC, num_subcores=NS
    )

    @functools.partial(
        pl.kernel,
        mesh=mesh,
        compiler_params=pltpu.CompilerParams(
            use_tc_tiling_on_sc=False, needs_layout_passes=False
        ),
        out_type=jax.ShapeDtypeStruct((SEQ, 8, 8, 8, PADDED), jnp.float32),
        scratch_types=[
            pltpu.VMEM((MAXTR, 8, PADDED), jnp.int32),
            pltpu.VMEM((MAXTR * 8 * PADDED,), jnp.int32),
            pltpu.VMEM((SEQ, EMB), jnp.float32),
            pltpu.VMEM((ROWS, EMB), jnp.float32),
            pltpu.VMEM((ROWS, EMB), jnp.float32),
            pltpu.VMEM((S_CHUNK, 8, 8, PADDED + 1), jnp.float32),
            pltpu.VMEM((S_CHUNK, 8, 8, PADDED + 1), jnp.float32),
            pltpu.SemaphoreType.DMA,
            pltpu.SemaphoreType.DMA,
            pltpu.SemaphoreType.DMA,
            pltpu.SemaphoreType.DMA,
            pltpu.SemaphoreType.DMA,
        ],
    )
    def k(idx_hbm, tok_hbm, pos_hbm, out_hbm, idx_st, idx2_v, pos_v,
          gb0, gb1, ob0, ob1, ssem, gs0, gs1, ws0, ws1):
        gbufs = (gb0, gb1)
        obufs = (ob0, ob1)
        gsem = (gs0, gs1)
        wsem = (ws0, ws1)
        wid = lax.axis_index("s") * NC + lax.axis_index("c")
        tb = wid // 4
        q = wid % 4
        ntr = jnp.where(q == 0, 7, 6)
        tr0 = jnp.where(q == 0, 0, 7 + (q - 1) * 6)
        nch = ntr * 4

        pltpu.sync_copy(pos_hbm, pos_v)

        def idx_dma(t, carry):
            pltpu.async_copy(idx_hbm.at[tr0 + t, tb], idx_st.at[t], ssem)
            return carry

        lax.fori_loop(0, ntr, idx_dma, 0)

        def drain_idx(t, carry):
            pltpu.make_async_copy(idx_hbm.at[0, 0], idx_st.at[0], ssem).wait()
            return carry

        lax.fori_loop(0, ntr, drain_idx, 0)

        def idx_flat(t, carry):
            for sl in range(8):
                for h in range(PADDED // LANES):
                    off = t * 1024 + sl * PADDED + h * LANES
                    idx2_v[pl.ds(off, LANES)] = (
                        idx_st[t, sl, pl.ds(h * LANES, LANES)] * 2
                    )
            return carry

        lax.fori_loop(0, ntr, idx_flat, 0)

        def start_gather(kc, b):
            for h in range(2):
                pltpu.async_copy(
                    tok_hbm.at[idx2_v.at[pl.ds(kc * ROWS + h * 128, 128)]],
                    gbufs[b].at[pl.ds(h * 128, 128)],
                    gsem[b],
                )

        def wait_gather(b):
            pltpu.make_async_copy(tok_hbm.at[pl.ds(0, ROWS)], gbufs[b], gsem[b]).wait()

        def start_wb(kc, b):
            s0 = (tr0 + kc // 4) * 8 + (kc % 4) * S_CHUNK
            pltpu.async_copy(
                obufs[b].at[:, :, :, pl.ds(0, PADDED)],
                out_hbm.at[pl.ds(s0, S_CHUNK), :, tb, :, :],
                wsem[b],
            )

        def wait_wb(b):
            pltpu.make_async_copy(
                obufs[b].at[:, :, :, pl.ds(0, PADDED)],
                out_hbm.at[pl.ds(0, S_CHUNK), :, 0, :, :],
                wsem[b],
            ).wait()

        iota = lax.iota(jnp.int32, LANES)
        c_td = [(j * LANES + iota) >> 3 for j in range(EMB // LANES)]
        c_sd = [(j * LANES + iota) & 7 for j in range(EMB // LANES)]

        def compute(kc, b):
            gbuf = gbufs[b]
            obuf = obufs[b]
            s0 = (tr0 + kc // 4) * 8 + (kc % 4) * S_CHUNK
            for ss in range(S_CHUNK):
                prow = s0 + ss
                pos_j = [
                    pos_v[prow, pl.ds(j * LANES, LANES)]
                    for j in range(EMB // LANES)
                ]
                c_ss = jnp.full((LANES,), ss, jnp.int32)

                def b_body(bq, carry):
                    for u in range(4):
                        bv = bq * 4 + u
                        row = ss * PADDED + bv
                        i_b = jnp.full((LANES,), bv, jnp.int32)
                        for j in range(EMB // LANES):
                            v = gbuf[row, pl.ds(j * LANES, LANES)] * SCALE \
                                + pos_j[j]
                            plsc.store_scatter(
                                obuf, [c_ss, c_td[j], c_sd[j], i_b], v
                            )
                    return carry

                lax.fori_loop(0, PADDED // 4, b_body, 0)

        start_gather(0, 0)

        def outer(o, carry):
            for phase in range(2):
                kc = 2 * o + phase
                b = phase

                @pl.when(kc + 1 < nch)
                def _():
                    start_gather(kc + 1, 1 - phase)

                wait_gather(b)

                @pl.when(kc >= 2)
                def _():
                    wait_wb(b)

                compute(kc, b)
                start_wb(kc, b)
            return carry

        lax.fori_loop(0, nch // 2, outer, 0)
        wait_wb(0)
        wait_wb(1)

    return k(idx4, tblv, pos_table)


def kernel(inputs, token_table, pos_table):
    idx4 = (
        inputs.astype(jnp.int32)
        .T.reshape(STR, 8, 8, PADDED)
        .transpose(0, 2, 1, 3)
    )
    tbl128 = jnp.pad(token_table.astype(jnp.float32), ((0, 0), (0, PADDED - EMB)))
    tblv = tbl128.reshape(2 * VOCAB, EMB)
    out5 = _sc_embed(idx4, tblv, pos_table.astype(jnp.float32))
    return jnp.transpose(out5, (2, 4, 0, 1, 3)).reshape(BATCH, SEQ, EMB)
